# baseline (speedup 1.0000x reference)
import functools
import math

import jax
import jax.numpy as jnp
from jax import lax
from jax.experimental import pallas as pl
from jax.experimental.pallas import tpu as pltpu
from jax.experimental.pallas import tpu_sc as plsc

_CAPACITY_FACTOR = 2.0
_MIN_CAPACITY = 4


def _capacity(s: int, e: int) -> int:
    c = math.floor(_CAPACITY_FACTOR * s / e)
    c += c % 2
    return max(c, _MIN_CAPACITY)


def _sc_top2_body(x_hbm, w_hbm, r_hbm, xv, wpl, rpl, sem, *, tok_w):
    E = 16
    wid = lax.axis_index("s")
    base = wid * tok_w
    pltpu.sync_copy(x_hbm.at[pl.ds(base, tok_w)], xv)

    iota = lax.broadcasted_iota(jnp.int32, (E,), 0)
    zf = jnp.zeros((E,), jnp.float32)
    zi = jnp.zeros((E,), jnp.int32)
    neg = jnp.full((E,), -3.0e38, jnp.float32)
    onef = jnp.ones((E,), jnp.float32)
    onei = jnp.ones((E,), jnp.int32)
    ev = jnp.full((E,), E, jnp.int32)
    c256 = jnp.full((E,), 256, jnp.int32)
    neg1 = jnp.full((E,), -1, jnp.int32)

    def _shuf(v, k):
        return v.at[iota ^ k].get(mode="promise_in_bounds")

    def _bmax(v):
        for k in (1, 2, 4, 8):
            v = jnp.maximum(v, _shuf(v, k))
        return v

    def _bmin(v):
        for k in (1, 2, 4, 8):
            v = jnp.minimum(v, _shuf(v, k))
        return v

    def _bsum(v):
        for k in (1, 2, 4, 8):
            v = v + _shuf(v, k)
        return v

    def pass_a(t, carry):
        c1, c2 = carry
        row = xv[t]
        m1 = _bmax(row)
        e1 = _bmin(jnp.where(row == m1, iota, ev))
        oh1 = iota == e1
        row2 = jnp.where(oh1, neg, row)
        m2 = _bmax(row2)
        e2 = _bmin(jnp.where(row2 == m2, iota, ev))
        oh2 = iota == e2
        u = jnp.exp(row - m1)
        sv = _bsum(u)
        w1 = onef / sv
        w2 = u.at[e2].get(mode="promise_in_bounds") / sv
        lr1 = c1.at[e1].get(mode="promise_in_bounds")
        lr2 = c2.at[e2].get(mode="promise_in_bounds")
        wpl[t] = jnp.where(oh1, w1, jnp.where(oh2, w2, zf))
        rpl[t] = jnp.where(oh1, lr1, jnp.where(oh2, lr2 + c256, neg1))
        return (c1 + jnp.where(oh1, onei, zi), c2 + jnp.where(oh2, onei, zi))

    lax.fori_loop(0, tok_w, pass_a, (zi, zi))
    pltpu.sync_copy(wpl, w_hbm.at[pl.ds(base, tok_w)])
    pltpu.sync_copy(rpl, r_hbm.at[pl.ds(base, tok_w)])


def _tc_build_kernel(w_ref, r_ref, cb_ref, mask_ref, w_s, r_s, *, cap, nw):
    pid = pl.program_id(0)
    S, E = w_ref.shape
    T, _, C = cb_ref.shape
    seg = S // nw

    @pl.when(pid == 0)
    def _globalize():
        wl = w_ref[...]
        rp = r_ref[...]
        ise1 = (rp >= 0) & (rp < 256)
        ise2 = rp >= 256
        rl = jnp.where(ise2, rp - 256, rp)
        r3 = rl.reshape(nw, seg, E)
        m1 = jnp.where(ise1.reshape(nw, seg, E), r3 + 1, 0)
        m2 = jnp.where(ise2.reshape(nw, seg, E), r3 + 1, 0)
        cnt1 = jnp.max(m1, axis=1)
        cnt2 = jnp.max(m2, axis=1)
        def _excl_cumsum(v):
            k = 1
            out = v
            while k < nw:
                out = out + jnp.concatenate(
                    [jnp.zeros((k, E), v.dtype), out[: nw - k, :]], axis=0)
                k *= 2
            return out - v

        pref1 = _excl_cumsum(cnt1)
        pref2 = _excl_cumsum(cnt2)
        tot1 = jnp.sum(cnt1, axis=0, keepdims=True)
        off1 = jnp.broadcast_to(pref1[:, None, :], (nw, seg, E)).reshape(S, E)
        off2 = jnp.broadcast_to(
            (pref2 + tot1)[:, None, :], (nw, seg, E)).reshape(S, E)
        rank = jnp.where(ise1, rl + off1, jnp.where(ise2, rl + off2, -1))
        keep = (rank >= 0) & (rank < cap)
        w = jnp.where(keep, wl, 0.0)
        r = jnp.where(keep & (w != 0.0), rank, -1)
        w_s[...] = w
        r_s[...] = r

    t0 = pid * T
    w = w_s[pl.ds(t0, T), :]
    r = r_s[pl.ds(t0, T), :]
    c_iota = lax.broadcasted_iota(jnp.int32, (T, E, C), 2)
    eq = c_iota == r[:, :, None]
    cb_ref[...] = jnp.where(eq, w[:, :, None], 0.0)
    mask_ref[...] = eq


def kernel(inputs):
    S, E = inputs.shape
    C = _capacity(S, E)
    x = inputs.astype(jnp.float32)

    ns = plsc.get_sparse_core_info().num_subcores
    tok_w = S // ns
    mesh = plsc.VectorSubcoreMesh(
        core_axis_name="c", subcore_axis_name="s", num_cores=1)
    wl, rp = pl.kernel(
        functools.partial(_sc_top2_body, tok_w=tok_w),
        mesh=mesh,
        out_type=[
            jax.ShapeDtypeStruct((S, E), jnp.float32),
            jax.ShapeDtypeStruct((S, E), jnp.int32),
        ],
        scratch_types=[
            pltpu.VMEM((tok_w, E), jnp.float32),
            pltpu.VMEM((tok_w, E), jnp.float32),
            pltpu.VMEM((tok_w, E), jnp.int32),
            pltpu.SemaphoreType.DMA,
        ],
    )(x)

    T = 256
    cb, mask = pl.pallas_call(
        functools.partial(_tc_build_kernel, cap=C, nw=ns),
        grid=(S // T,),
        in_specs=[
            pl.BlockSpec((S, E), lambda i: (0, 0)),
            pl.BlockSpec((S, E), lambda i: (0, 0)),
        ],
        out_specs=[
            pl.BlockSpec((T, E, C), lambda i: (i, 0, 0)),
            pl.BlockSpec((T, E, C), lambda i: (i, 0, 0)),
        ],
        out_shape=[
            jax.ShapeDtypeStruct((S, E, C), jnp.float32),
            jax.ShapeDtypeStruct((S, E, C), jnp.bool_),
        ],
        scratch_shapes=[
            pltpu.VMEM((S, E), jnp.float32),
            pltpu.VMEM((S, E), jnp.int32),
        ],
    )(wl, rp)
    return (cb, mask)


# speedup vs baseline: 1.0040x; 1.0040x over previous
import functools
import math

import jax
import jax.numpy as jnp
from jax import lax
from jax.experimental import pallas as pl
from jax.experimental.pallas import tpu as pltpu
from jax.experimental.pallas import tpu_sc as plsc

_CAPACITY_FACTOR = 2.0
_MIN_CAPACITY = 4


def _capacity(s: int, e: int) -> int:
    c = math.floor(_CAPACITY_FACTOR * s / e)
    c += c % 2
    return max(c, _MIN_CAPACITY)


def _sc_top2_body(x_hbm, w_hbm, r_hbm, xv, wpl, rpl, sem, *, tok_w, nc):
    E = 16
    wid = lax.axis_index("s") * nc + lax.axis_index("c")
    base = wid * tok_w
    pltpu.sync_copy(x_hbm.at[pl.ds(base, tok_w)], xv)

    iota = lax.broadcasted_iota(jnp.int32, (E,), 0)
    zf = jnp.zeros((E,), jnp.float32)
    zi = jnp.zeros((E,), jnp.int32)
    neg = jnp.full((E,), -3.0e38, jnp.float32)
    onef = jnp.ones((E,), jnp.float32)
    onei = jnp.ones((E,), jnp.int32)
    ev = jnp.full((E,), E, jnp.int32)
    c256 = jnp.full((E,), 256, jnp.int32)
    neg1 = jnp.full((E,), -1, jnp.int32)

    def _shuf(v, k):
        return v.at[iota ^ k].get(mode="promise_in_bounds")

    def _bmax(v):
        for k in (1, 2, 4, 8):
            v = jnp.maximum(v, _shuf(v, k))
        return v

    def _bmin(v):
        for k in (1, 2, 4, 8):
            v = jnp.minimum(v, _shuf(v, k))
        return v

    def _bsum(v):
        for k in (1, 2, 4, 8):
            v = v + _shuf(v, k)
        return v

    def pass_a(t, carry):
        c1, c2 = carry
        row = xv[t]
        m1 = _bmax(row)
        e1 = _bmin(jnp.where(row == m1, iota, ev))
        oh1 = iota == e1
        row2 = jnp.where(oh1, neg, row)
        m2 = _bmax(row2)
        e2 = _bmin(jnp.where(row2 == m2, iota, ev))
        oh2 = iota == e2
        u = jnp.exp(row - m1)
        sv = _bsum(u)
        w1 = onef / sv
        w2 = u.at[e2].get(mode="promise_in_bounds") / sv
        lr1 = c1.at[e1].get(mode="promise_in_bounds")
        lr2 = c2.at[e2].get(mode="promise_in_bounds")
        wpl[t] = jnp.where(oh1, w1, jnp.where(oh2, w2, zf))
        rpl[t] = jnp.where(oh1, lr1, jnp.where(oh2, lr2 + c256, neg1))
        return (c1 + jnp.where(oh1, onei, zi), c2 + jnp.where(oh2, onei, zi))

    lax.fori_loop(0, tok_w, pass_a, (zi, zi))
    pltpu.sync_copy(wpl, w_hbm.at[pl.ds(base, tok_w)])
    pltpu.sync_copy(rpl, r_hbm.at[pl.ds(base, tok_w)])


def _tc_build_kernel(w_ref, r_ref, cb_ref, mask_ref, w_s, r_s, *, cap, nw):
    pid = pl.program_id(0)
    S, E = w_ref.shape
    T, _, C = cb_ref.shape
    seg = S // nw

    @pl.when(pid == 0)
    def _globalize():
        wl = w_ref[...]
        rp = r_ref[...]
        ise1 = (rp >= 0) & (rp < 256)
        ise2 = rp >= 256
        rl = jnp.where(ise2, rp - 256, rp)
        r3 = rl.reshape(nw, seg, E)
        m1 = jnp.where(ise1.reshape(nw, seg, E), r3 + 1, 0)
        m2 = jnp.where(ise2.reshape(nw, seg, E), r3 + 1, 0)
        cnt1 = jnp.max(m1, axis=1)
        cnt2 = jnp.max(m2, axis=1)
        def _excl_cumsum(v):
            k = 1
            out = v
            while k < nw:
                out = out + jnp.concatenate(
                    [jnp.zeros((k, E), v.dtype), out[: nw - k, :]], axis=0)
                k *= 2
            return out - v

        pref1 = _excl_cumsum(cnt1)
        pref2 = _excl_cumsum(cnt2)
        tot1 = jnp.sum(cnt1, axis=0, keepdims=True)
        off1 = jnp.broadcast_to(pref1[:, None, :], (nw, seg, E)).reshape(S, E)
        off2 = jnp.broadcast_to(
            (pref2 + tot1)[:, None, :], (nw, seg, E)).reshape(S, E)
        rank = jnp.where(ise1, rl + off1, jnp.where(ise2, rl + off2, -1))
        keep = (rank >= 0) & (rank < cap)
        w = jnp.where(keep, wl, 0.0)
        r = jnp.where(keep & (w != 0.0), rank, -1)
        w_s[...] = w
        r_s[...] = r

    t0 = pid * T
    w = w_s[pl.ds(t0, T), :]
    r = r_s[pl.ds(t0, T), :]
    c_iota = lax.broadcasted_iota(jnp.int32, (T, E, C), 2)
    eq = c_iota == r[:, :, None]
    cb_ref[...] = jnp.where(eq, w[:, :, None], 0.0)
    mask_ref[...] = eq


def kernel(inputs):
    S, E = inputs.shape
    C = _capacity(S, E)
    x = inputs.astype(jnp.float32)

    info = plsc.get_sparse_core_info()
    nw = info.num_cores * info.num_subcores
    tok_w = S // nw
    mesh = plsc.VectorSubcoreMesh(core_axis_name="c", subcore_axis_name="s")
    wl, rp = pl.kernel(
        functools.partial(_sc_top2_body, tok_w=tok_w, nc=info.num_cores),
        mesh=mesh,
        out_type=[
            jax.ShapeDtypeStruct((S, E), jnp.float32),
            jax.ShapeDtypeStruct((S, E), jnp.int32),
        ],
        scratch_types=[
            pltpu.VMEM((tok_w, E), jnp.float32),
            pltpu.VMEM((tok_w, E), jnp.float32),
            pltpu.VMEM((tok_w, E), jnp.int32),
            pltpu.SemaphoreType.DMA,
        ],
    )(x)

    T = 256
    cb, mask = pl.pallas_call(
        functools.partial(_tc_build_kernel, cap=C, nw=nw),
        grid=(S // T,),
        in_specs=[
            pl.BlockSpec((S, E), lambda i: (0, 0)),
            pl.BlockSpec((S, E), lambda i: (0, 0)),
        ],
        out_specs=[
            pl.BlockSpec((T, E, C), lambda i: (i, 0, 0)),
            pl.BlockSpec((T, E, C), lambda i: (i, 0, 0)),
        ],
        out_shape=[
            jax.ShapeDtypeStruct((S, E, C), jnp.float32),
            jax.ShapeDtypeStruct((S, E, C), jnp.bool_),
        ],
        scratch_shapes=[
            pltpu.VMEM((S, E), jnp.float32),
            pltpu.VMEM((S, E), jnp.int32),
        ],
    )(wl, rp)
    return (cb, mask)
